# baseline (speedup 1.0000x reference)
import functools

import jax
import jax.numpy as jnp
from jax import lax
from jax.experimental import pallas as pl
from jax.experimental.pallas import tpu as pltpu
from jax.experimental.pallas import tpu_sc as plsc

_N = 10000
_E = 320000
_G = 64
_D = 128
_OUT = 16
_LAYERS = 5
_BNEPS = 1e-5

_NC, _NS = 2, 16
_NW = _NC * _NS
_CHUNK = 80
_CPT = 128
_EPAD = _NW * _CPT * _CHUNK
_ZR = 640
_NPAD = _NW * _ZR // _NC
_WR = _ZR

_RING = 4
_BPW = _CPT // _RING
_BE = _RING * _CHUNK

_BLK = 1000
_NBLK = _N // _BLK


def _scale_chunk(rows_v, w_v, k):
    for g in range(_CHUNK // 16):
        wv = w_v[pl.ds(k * _CHUNK + g * 16, 16)]
        wes = [wv[e] for e in range(16)]
        for e in range(16):
            r = g * 16 + e
            for f in range(_D // 16):
                sl = pl.ds(f * 16, 16)
                rows_v[r, sl] = rows_v[r, sl] * wes[e]


def _mp_body(h_hbm, swd_hbm, w_hbm, zeros_hbm, out_hbm,
             swd0, swd1, w0, w1, r0, r1, r2, r3,
             sem0, sem1, sem2, sem3, acc_sh):
    c = lax.axis_index("c")
    s = lax.axis_index("s")
    wid = s * _NC + c
    bbase = wid * _BPW

    pltpu.sync_copy(zeros_hbm, acc_sh.at[pl.ds(s * _ZR, _ZR)])
    plsc.subcore_barrier()

    rbufs = (r0, r1, r2, r3)
    sems = (sem0, sem1, sem2, sem3)
    sbufs = (swd0, swd1)
    wbufs = (w0, w1)

    def src_ref(sb, k):
        return h_hbm.at[sb.at[pl.ds(k * _CHUNK, _CHUNK)]]

    def load_block(bi, q):
        pltpu.sync_copy(swd_hbm.at[bbase + bi], sbufs[q])
        pltpu.sync_copy(w_hbm.at[pl.ds((bbase + bi) * _BE, _BE)], wbufs[q])

    load_block(0, 0)
    for k in range(_RING):
        pltpu.async_copy(src_ref(swd0, k), rbufs[k], sems[k])
    load_block(1, 1)

    def body(i, carry):
        for q in range(2):
            b = 2 * i + q
            sb = sbufs[q]
            nb = sbufs[1 - q]
            for k in range(_RING):
                j = _RING * b + k
                rows_v = rbufs[k]
                pltpu.make_async_copy(src_ref(sb, k), rows_v, sems[k]).wait()
                _scale_chunk(rows_v, wbufs[q], k)
                pltpu.sync_copy(
                    rows_v,
                    acc_sh.at[sb.at[pl.ds(_BE + k * _CHUNK, _CHUNK)]],
                    add=True)

                @pl.when(j + _RING < _CPT)
                def _():
                    pltpu.async_copy(src_ref(nb, k), rows_v, sems[k])

            @pl.when(b + 2 < _BPW)
            def _():
                load_block(b + 2, q)
        return carry

    lax.fori_loop(0, _BPW // 2, body, 0)
    plsc.subcore_barrier()
    pltpu.sync_copy(acc_sh.at[pl.ds(s * _WR, _WR)],
                    out_hbm.at[c].at[pl.ds(s * _WR, _WR)])


@functools.cache
def _mp_builder():
    return functools.partial(
        pl.kernel,
        out_type=jax.ShapeDtypeStruct((_NC, _NPAD, _D), jnp.float32),
        mesh=plsc.VectorSubcoreMesh(core_axis_name="c", subcore_axis_name="s",
                                    num_cores=_NC, num_subcores=_NS),
        scratch_types=[
            pltpu.VMEM((2 * _BE,), jnp.int32),
            pltpu.VMEM((2 * _BE,), jnp.int32),
            pltpu.VMEM((_BE,), jnp.float32),
            pltpu.VMEM((_BE,), jnp.float32),
            pltpu.VMEM((_CHUNK, _D), jnp.float32),
            pltpu.VMEM((_CHUNK, _D), jnp.float32),
            pltpu.VMEM((_CHUNK, _D), jnp.float32),
            pltpu.VMEM((_CHUNK, _D), jnp.float32),
            pltpu.SemaphoreType.DMA,
            pltpu.SemaphoreType.DMA,
            pltpu.SemaphoreType.DMA,
            pltpu.SemaphoreType.DMA,
            pltpu.VMEM_SHARED((_NPAD, _D), jnp.float32),
        ],
    )(_mp_body)


def _p1_body(heps_ref, h_ref, hn_ref, w1_ref, b1_ref, t_ref, st_ref):
    i = pl.program_id(0)
    a = h_ref[...] * heps_ref[0, 0] + hn_ref[0] + hn_ref[1]
    t = jnp.dot(a, w1_ref[...], preferred_element_type=jnp.float32)
    t = t + b1_ref[...]
    t_ref[...] = t

    @pl.when(i == 0)
    def _():
        st_ref[...] = jnp.zeros_like(st_ref)

    st_ref[0:1] += jnp.sum(t, axis=0, keepdims=True)
    st_ref[1:2] += jnp.sum(t * t, axis=0, keepdims=True)


def _p1(heps, h, hn, w1, b1):
    return pl.pallas_call(
        _p1_body,
        grid=(_NBLK,),
        in_specs=[
            pl.BlockSpec(memory_space=pltpu.SMEM),
            pl.BlockSpec((_BLK, _D), lambda i: (i, 0)),
            pl.BlockSpec((_NC, _BLK, _D), lambda i: (0, i, 0)),
            pl.BlockSpec((_D, _D), lambda i: (0, 0)),
            pl.BlockSpec((1, _D), lambda i: (0, 0)),
        ],
        out_specs=[
            pl.BlockSpec((_BLK, _D), lambda i: (i, 0)),
            pl.BlockSpec((8, _D), lambda i: (0, 0)),
        ],
        out_shape=[
            jax.ShapeDtypeStruct((_N, _D), jnp.float32),
            jax.ShapeDtypeStruct((8, _D), jnp.float32),
        ],
    )(heps, h, hn, w1, b1)


def _p2_body(st_ref, g_ref, be_ref, t_ref, w2_ref, b2_ref, u_ref, st2_ref):
    i = pl.program_id(0)
    mean = st_ref[0:1] * (1.0 / _N)
    var = st_ref[1:2] * (1.0 / _N) - mean * mean
    scale = g_ref[...] * lax.rsqrt(var + _BNEPS)
    shift = be_ref[...] - mean * scale
    r = jnp.maximum(t_ref[...] * scale + shift, 0.0)
    u = jnp.dot(r, w2_ref[...], preferred_element_type=jnp.float32)
    u = u + b2_ref[...]
    u_ref[...] = u

    @pl.when(i == 0)
    def _():
        st2_ref[...] = jnp.zeros_like(st2_ref)

    st2_ref[0:1] += jnp.sum(u, axis=0, keepdims=True)
    st2_ref[1:2] += jnp.sum(u * u, axis=0, keepdims=True)


def _p2(st, g, be, t, w2, b2):
    return pl.pallas_call(
        _p2_body,
        grid=(_NBLK,),
        in_specs=[
            pl.BlockSpec((8, _D), lambda i: (0, 0)),
            pl.BlockSpec((1, _D), lambda i: (0, 0)),
            pl.BlockSpec((1, _D), lambda i: (0, 0)),
            pl.BlockSpec((_BLK, _D), lambda i: (i, 0)),
            pl.BlockSpec((_D, _D), lambda i: (0, 0)),
            pl.BlockSpec((1, _D), lambda i: (0, 0)),
        ],
        out_specs=[
            pl.BlockSpec((_BLK, _D), lambda i: (i, 0)),
            pl.BlockSpec((8, _D), lambda i: (0, 0)),
        ],
        out_shape=[
            jax.ShapeDtypeStruct((_N, _D), jnp.float32),
            jax.ShapeDtypeStruct((8, _D), jnp.float32),
        ],
    )(st, g, be, t, w2, b2)


def _p3_body(st_ref, g_ref, be_ref, gid_ref, u_ref, h_ref, hg_ref):
    i = pl.program_id(0)
    mean = st_ref[0:1] * (1.0 / _N)
    var = st_ref[1:2] * (1.0 / _N) - mean * mean
    scale = g_ref[...] * lax.rsqrt(var + _BNEPS)
    shift = be_ref[...] - mean * scale
    h = jnp.maximum(u_ref[...] * scale + shift, 0.0)
    h_ref[...] = h
    oh = (lax.broadcasted_iota(jnp.int32, (_G, _BLK), 0)
          == gid_ref[0]).astype(jnp.float32)
    hgc = jnp.dot(oh, h, preferred_element_type=jnp.float32)

    @pl.when(i == 0)
    def _():
        hg_ref[...] = jnp.zeros_like(hg_ref)

    hg_ref[...] += hgc


def _p3(st, g, be, gid3, u):
    return pl.pallas_call(
        _p3_body,
        grid=(_NBLK,),
        in_specs=[
            pl.BlockSpec((8, _D), lambda i: (0, 0)),
            pl.BlockSpec((1, _D), lambda i: (0, 0)),
            pl.BlockSpec((1, _D), lambda i: (0, 0)),
            pl.BlockSpec((1, 1, _BLK), lambda i: (i, 0, 0)),
            pl.BlockSpec((_BLK, _D), lambda i: (i, 0)),
        ],
        out_specs=[
            pl.BlockSpec((_BLK, _D), lambda i: (i, 0)),
            pl.BlockSpec((_G, _D), lambda i: (0, 0)),
        ],
        out_shape=[
            jax.ShapeDtypeStruct((_N, _D), jnp.float32),
            jax.ShapeDtypeStruct((_G, _D), jnp.float32),
        ],
    )(st, g, be, gid3, u)


def _p0_body(gid_ref, x_ref, hg_ref):
    i = pl.program_id(0)
    oh = (lax.broadcasted_iota(jnp.int32, (_G, _BLK), 0)
          == gid_ref[0]).astype(jnp.float32)
    hgc = jnp.dot(oh, x_ref[...], preferred_element_type=jnp.float32)

    @pl.when(i == 0)
    def _():
        hg_ref[...] = jnp.zeros_like(hg_ref)

    hg_ref[...] += hgc


def _p0(gid3, x):
    return pl.pallas_call(
        _p0_body,
        grid=(_NBLK,),
        in_specs=[
            pl.BlockSpec((1, 1, _BLK), lambda i: (i, 0, 0)),
            pl.BlockSpec((_BLK, _D), lambda i: (i, 0)),
        ],
        out_specs=pl.BlockSpec((_G, _D), lambda i: (0, 0)),
        out_shape=jax.ShapeDtypeStruct((_G, _D), jnp.float32),
    )(gid3, x)


def _sk_body(hg_ref, pw_ref, pb_ref, score_ref):
    acc = jnp.zeros((_G, _OUT), jnp.float32)
    for l in range(_LAYERS):
        acc = acc + jnp.dot(hg_ref[l], pw_ref[l],
                            preferred_element_type=jnp.float32) + pb_ref[l]
    score_ref[...] = acc


def _sk(hg_all, pw, pb):
    return pl.pallas_call(
        _sk_body,
        out_shape=jax.ShapeDtypeStruct((_G, _OUT), jnp.float32),
    )(hg_all, pw, pb)


@jax.jit
def kernel(x, edge_index, edge_weight, graph_ids, params):
    pad = _EPAD - _E
    src = jnp.concatenate([edge_index[0], jnp.zeros((pad,), jnp.int32)])
    dst = jnp.concatenate([edge_index[1], jnp.zeros((pad,), jnp.int32)])
    w = jnp.concatenate([edge_weight, jnp.zeros((pad,), jnp.float32)])
    swd = jnp.concatenate([src.reshape(_NW * _BPW, _BE),
                          dst.reshape(_NW * _BPW, _BE)], axis=1)
    zeros_hbm = jnp.zeros((_ZR, _D), jnp.float32)
    gid3 = graph_ids.reshape(_NBLK, 1, _BLK)

    hgs = [_p0(gid3, x)]
    h = x
    for l in range(_LAYERS - 1):
        layer = params["layers"][l]
        hn = _mp_builder()(h, swd, w, zeros_hbm)
        heps = (1.0 + params["eps"][l]).reshape(1, 1)
        t, st1 = _p1(heps, h, hn, layer["W1"], layer["b1"].reshape(1, _D))
        u, st2 = _p2(st1, layer["bn1_gamma"].reshape(1, _D),
                     layer["bn1_beta"].reshape(1, _D), t,
                     layer["W2"], layer["b2"].reshape(1, _D))
        h, hg = _p3(st2, layer["bn_gamma"].reshape(1, _D),
                    layer["bn_beta"].reshape(1, _D), gid3, u)
        hgs.append(hg)

    hg_all = jnp.stack(hgs)
    pw = jnp.stack(params["pred_W"])
    pb = jnp.stack([b.reshape(1, _OUT) for b in params["pred_b"]])
    return _sk(hg_all, pw, pb)


# speedup vs baseline: 1.0086x; 1.0086x over previous
import functools

import jax
import jax.numpy as jnp
from jax import lax
from jax.experimental import pallas as pl
from jax.experimental.pallas import tpu as pltpu
from jax.experimental.pallas import tpu_sc as plsc

_N = 10000
_E = 320000
_G = 64
_D = 128
_OUT = 16
_LAYERS = 5
_BNEPS = 1e-5

_NC, _NS = 2, 16
_NW = _NC * _NS
_CHUNK = 80
_CPT = 128
_EPAD = _NW * _CPT * _CHUNK
_ZR = 640
_NPAD = _NW * _ZR // _NC
_WR = _ZR

_RING = 4
_BPW = _CPT // _RING
_BE = _RING * _CHUNK

_BLK = 1000
_NBLK = _N // _BLK


def _scale_chunk(rows_v, w_v, k):
    for g in range(_CHUNK // 16):
        wv = w_v[pl.ds(k * _CHUNK + g * 16, 16)]
        wes = [wv[e] for e in range(16)]
        for e in range(16):
            r = g * 16 + e
            for f in range(_D // 16):
                sl = pl.ds(f * 16, 16)
                rows_v[r, sl] = rows_v[r, sl] * wes[e]


def _mp_body(h_hbm, swd_hbm, w_hbm, zeros_hbm, out_hbm,
             swd0, swd1, w0, w1, r0, r1, r2, r3,
             sem0, sem1, sem2, sem3,
             d0, d1, d2, d3, ssem0, ssem1, ssem2, ssem3, acc_sh):
    c = lax.axis_index("c")
    s = lax.axis_index("s")
    wid = s * _NC + c
    bbase = wid * _BPW

    pltpu.sync_copy(zeros_hbm, acc_sh.at[pl.ds(s * _ZR, _ZR)])
    plsc.subcore_barrier()

    rbufs = (r0, r1, r2, r3)
    sems = (sem0, sem1, sem2, sem3)
    dbufs = (d0, d1, d2, d3)
    ssems = (ssem0, ssem1, ssem2, ssem3)
    sbufs = (swd0, swd1)
    wbufs = (w0, w1)

    def src_ref(sb, k):
        return h_hbm.at[sb.at[pl.ds(k * _CHUNK, _CHUNK)]]

    def scat_copy(k):
        return pltpu.make_async_copy(rbufs[k], acc_sh.at[dbufs[k]], ssems[k])

    def load_block(bi, q):
        pltpu.sync_copy(swd_hbm.at[bbase + bi], sbufs[q])
        pltpu.sync_copy(w_hbm.at[pl.ds((bbase + bi) * _BE, _BE)], wbufs[q])

    load_block(0, 0)
    for k in range(2):
        pltpu.async_copy(src_ref(swd0, k), rbufs[k], sems[k])
    load_block(1, 1)

    def body(i, carry):
        for q in range(2):
            b = 2 * i + q
            sb = sbufs[q]
            nb = sbufs[1 - q]
            for k in range(_RING):
                j = _RING * b + k
                rows_v = rbufs[k]
                pltpu.make_async_copy(src_ref(sb, k), rows_v, sems[k]).wait()
                _scale_chunk(rows_v, wbufs[q], k)
                for g in range(_CHUNK // 16):
                    sl = pl.ds(g * 16, 16)
                    dbufs[k][sl] = sb[pl.ds(_BE + k * _CHUNK + g * 16, 16)]
                pltpu.async_copy(rows_v, acc_sh.at[dbufs[k]], ssems[k],
                                 add=True)

                kc = (k + 2) % 4
                gb, ck = (sb, k + 2) if k < 2 else (nb, k - 2)

                @pl.when((j + 2 < _CPT) & (j >= 2))
                def _():
                    scat_copy(kc).wait()

                @pl.when(j + 2 < _CPT)
                def _():
                    pltpu.async_copy(src_ref(gb, ck), rbufs[kc], sems[kc])

            @pl.when(b + 2 < _BPW)
            def _():
                load_block(b + 2, q)
        return carry

    lax.fori_loop(0, _BPW // 2, body, 0)
    for k in range(_RING):
        scat_copy(k).wait()
    plsc.subcore_barrier()
    pltpu.sync_copy(acc_sh.at[pl.ds(s * _WR, _WR)],
                    out_hbm.at[c].at[pl.ds(s * _WR, _WR)])


@functools.cache
def _mp_builder():
    return functools.partial(
        pl.kernel,
        out_type=jax.ShapeDtypeStruct((_NC, _NPAD, _D), jnp.float32),
        mesh=plsc.VectorSubcoreMesh(core_axis_name="c", subcore_axis_name="s",
                                    num_cores=_NC, num_subcores=_NS),
        scratch_types=[
            pltpu.VMEM((2 * _BE,), jnp.int32),
            pltpu.VMEM((2 * _BE,), jnp.int32),
            pltpu.VMEM((_BE,), jnp.float32),
            pltpu.VMEM((_BE,), jnp.float32),
            pltpu.VMEM((_CHUNK, _D), jnp.float32),
            pltpu.VMEM((_CHUNK, _D), jnp.float32),
            pltpu.VMEM((_CHUNK, _D), jnp.float32),
            pltpu.VMEM((_CHUNK, _D), jnp.float32),
            pltpu.SemaphoreType.DMA,
            pltpu.SemaphoreType.DMA,
            pltpu.SemaphoreType.DMA,
            pltpu.SemaphoreType.DMA,
            pltpu.VMEM((_CHUNK,), jnp.int32),
            pltpu.VMEM((_CHUNK,), jnp.int32),
            pltpu.VMEM((_CHUNK,), jnp.int32),
            pltpu.VMEM((_CHUNK,), jnp.int32),
            pltpu.SemaphoreType.DMA,
            pltpu.SemaphoreType.DMA,
            pltpu.SemaphoreType.DMA,
            pltpu.SemaphoreType.DMA,
            pltpu.VMEM_SHARED((_NPAD, _D), jnp.float32),
        ],
    )(_mp_body)


def _p1_body(heps_ref, h_ref, hn_ref, w1_ref, b1_ref, t_ref, st_ref):
    i = pl.program_id(0)
    a = h_ref[...] * heps_ref[0, 0] + hn_ref[0] + hn_ref[1]
    t = jnp.dot(a, w1_ref[...], preferred_element_type=jnp.float32)
    t = t + b1_ref[...]
    t_ref[...] = t

    @pl.when(i == 0)
    def _():
        st_ref[...] = jnp.zeros_like(st_ref)

    st_ref[0:1] += jnp.sum(t, axis=0, keepdims=True)
    st_ref[1:2] += jnp.sum(t * t, axis=0, keepdims=True)


def _p1(heps, h, hn, w1, b1):
    return pl.pallas_call(
        _p1_body,
        grid=(_NBLK,),
        in_specs=[
            pl.BlockSpec(memory_space=pltpu.SMEM),
            pl.BlockSpec((_BLK, _D), lambda i: (i, 0)),
            pl.BlockSpec((_NC, _BLK, _D), lambda i: (0, i, 0)),
            pl.BlockSpec((_D, _D), lambda i: (0, 0)),
            pl.BlockSpec((1, _D), lambda i: (0, 0)),
        ],
        out_specs=[
            pl.BlockSpec((_BLK, _D), lambda i: (i, 0)),
            pl.BlockSpec((8, _D), lambda i: (0, 0)),
        ],
        out_shape=[
            jax.ShapeDtypeStruct((_N, _D), jnp.float32),
            jax.ShapeDtypeStruct((8, _D), jnp.float32),
        ],
    )(heps, h, hn, w1, b1)


def _p2_body(st_ref, g_ref, be_ref, t_ref, w2_ref, b2_ref, u_ref, st2_ref):
    i = pl.program_id(0)
    mean = st_ref[0:1] * (1.0 / _N)
    var = st_ref[1:2] * (1.0 / _N) - mean * mean
    scale = g_ref[...] * lax.rsqrt(var + _BNEPS)
    shift = be_ref[...] - mean * scale
    r = jnp.maximum(t_ref[...] * scale + shift, 0.0)
    u = jnp.dot(r, w2_ref[...], preferred_element_type=jnp.float32)
    u = u + b2_ref[...]
    u_ref[...] = u

    @pl.when(i == 0)
    def _():
        st2_ref[...] = jnp.zeros_like(st2_ref)

    st2_ref[0:1] += jnp.sum(u, axis=0, keepdims=True)
    st2_ref[1:2] += jnp.sum(u * u, axis=0, keepdims=True)


def _p2(st, g, be, t, w2, b2):
    return pl.pallas_call(
        _p2_body,
        grid=(_NBLK,),
        in_specs=[
            pl.BlockSpec((8, _D), lambda i: (0, 0)),
            pl.BlockSpec((1, _D), lambda i: (0, 0)),
            pl.BlockSpec((1, _D), lambda i: (0, 0)),
            pl.BlockSpec((_BLK, _D), lambda i: (i, 0)),
            pl.BlockSpec((_D, _D), lambda i: (0, 0)),
            pl.BlockSpec((1, _D), lambda i: (0, 0)),
        ],
        out_specs=[
            pl.BlockSpec((_BLK, _D), lambda i: (i, 0)),
            pl.BlockSpec((8, _D), lambda i: (0, 0)),
        ],
        out_shape=[
            jax.ShapeDtypeStruct((_N, _D), jnp.float32),
            jax.ShapeDtypeStruct((8, _D), jnp.float32),
        ],
    )(st, g, be, t, w2, b2)


def _p3_body(st_ref, g_ref, be_ref, gid_ref, u_ref, h_ref, hg_ref):
    i = pl.program_id(0)
    mean = st_ref[0:1] * (1.0 / _N)
    var = st_ref[1:2] * (1.0 / _N) - mean * mean
    scale = g_ref[...] * lax.rsqrt(var + _BNEPS)
    shift = be_ref[...] - mean * scale
    h = jnp.maximum(u_ref[...] * scale + shift, 0.0)
    h_ref[...] = h
    oh = (lax.broadcasted_iota(jnp.int32, (_G, _BLK), 0)
          == gid_ref[0]).astype(jnp.float32)
    hgc = jnp.dot(oh, h, preferred_element_type=jnp.float32)

    @pl.when(i == 0)
    def _():
        hg_ref[...] = jnp.zeros_like(hg_ref)

    hg_ref[...] += hgc


def _p3(st, g, be, gid3, u):
    return pl.pallas_call(
        _p3_body,
        grid=(_NBLK,),
        in_specs=[
            pl.BlockSpec((8, _D), lambda i: (0, 0)),
            pl.BlockSpec((1, _D), lambda i: (0, 0)),
            pl.BlockSpec((1, _D), lambda i: (0, 0)),
            pl.BlockSpec((1, 1, _BLK), lambda i: (i, 0, 0)),
            pl.BlockSpec((_BLK, _D), lambda i: (i, 0)),
        ],
        out_specs=[
            pl.BlockSpec((_BLK, _D), lambda i: (i, 0)),
            pl.BlockSpec((_G, _D), lambda i: (0, 0)),
        ],
        out_shape=[
            jax.ShapeDtypeStruct((_N, _D), jnp.float32),
            jax.ShapeDtypeStruct((_G, _D), jnp.float32),
        ],
    )(st, g, be, gid3, u)


def _p0_body(gid_ref, x_ref, hg_ref):
    i = pl.program_id(0)
    oh = (lax.broadcasted_iota(jnp.int32, (_G, _BLK), 0)
          == gid_ref[0]).astype(jnp.float32)
    hgc = jnp.dot(oh, x_ref[...], preferred_element_type=jnp.float32)

    @pl.when(i == 0)
    def _():
        hg_ref[...] = jnp.zeros_like(hg_ref)

    hg_ref[...] += hgc


def _p0(gid3, x):
    return pl.pallas_call(
        _p0_body,
        grid=(_NBLK,),
        in_specs=[
            pl.BlockSpec((1, 1, _BLK), lambda i: (i, 0, 0)),
            pl.BlockSpec((_BLK, _D), lambda i: (i, 0)),
        ],
        out_specs=pl.BlockSpec((_G, _D), lambda i: (0, 0)),
        out_shape=jax.ShapeDtypeStruct((_G, _D), jnp.float32),
    )(gid3, x)


def _sk_body(hg_ref, pw_ref, pb_ref, score_ref):
    acc = jnp.zeros((_G, _OUT), jnp.float32)
    for l in range(_LAYERS):
        acc = acc + jnp.dot(hg_ref[l], pw_ref[l],
                            preferred_element_type=jnp.float32) + pb_ref[l]
    score_ref[...] = acc


def _sk(hg_all, pw, pb):
    return pl.pallas_call(
        _sk_body,
        out_shape=jax.ShapeDtypeStruct((_G, _OUT), jnp.float32),
    )(hg_all, pw, pb)


@jax.jit
def kernel(x, edge_index, edge_weight, graph_ids, params):
    pad = _EPAD - _E
    src = jnp.concatenate([edge_index[0], jnp.zeros((pad,), jnp.int32)])
    dst = jnp.concatenate([edge_index[1], jnp.zeros((pad,), jnp.int32)])
    w = jnp.concatenate([edge_weight, jnp.zeros((pad,), jnp.float32)])
    swd = jnp.concatenate([src.reshape(_NW * _BPW, _BE),
                          dst.reshape(_NW * _BPW, _BE)], axis=1)
    zeros_hbm = jnp.zeros((_ZR, _D), jnp.float32)
    gid3 = graph_ids.reshape(_NBLK, 1, _BLK)

    hgs = [_p0(gid3, x)]
    h = x
    for l in range(_LAYERS - 1):
        layer = params["layers"][l]
        hn = _mp_builder()(h, swd, w, zeros_hbm)
        heps = (1.0 + params["eps"][l]).reshape(1, 1)
        t, st1 = _p1(heps, h, hn, layer["W1"], layer["b1"].reshape(1, _D))
        u, st2 = _p2(st1, layer["bn1_gamma"].reshape(1, _D),
                     layer["bn1_beta"].reshape(1, _D), t,
                     layer["W2"], layer["b2"].reshape(1, _D))
        h, hg = _p3(st2, layer["bn_gamma"].reshape(1, _D),
                    layer["bn_beta"].reshape(1, _D), gid3, u)
        hgs.append(hg)

    hg_all = jnp.stack(hgs)
    pw = jnp.stack(params["pred_W"])
    pb = jnp.stack([b.reshape(1, _OUT) for b in params["pred_b"]])
    return _sk(hg_all, pw, pb)


# speedup vs baseline: 1.0323x; 1.0235x over previous
import functools

import jax
import jax.numpy as jnp
from jax import lax
from jax.experimental import pallas as pl
from jax.experimental.pallas import tpu as pltpu
from jax.experimental.pallas import tpu_sc as plsc

_N = 10000
_E = 320000
_G = 64
_D = 128
_OUT = 16
_LAYERS = 5
_BNEPS = 1e-5

_NC, _NS = 2, 16
_NW = _NC * _NS
_CHUNK = 80
_CPT = 128
_EPAD = _NW * _CPT * _CHUNK
_ZR = 640
_NPAD = _NW * _ZR // _NC
_WR = _ZR

_RING = 4
_BPW = _CPT // _RING
_BE = _RING * _CHUNK

_BLK = 1000
_NBLK = _N // _BLK


def _scale_chunk(rows_v, w_v, k):
    for g in range(_CHUNK // 16):
        wv = w_v[pl.ds(k * _CHUNK + g * 16, 16)]
        wes = [wv[e] for e in range(16)]
        for e in range(16):
            r = g * 16 + e
            for f in range(_D // 16):
                sl = pl.ds(f * 16, 16)
                rows_v[r, sl] = rows_v[r, sl] * wes[e]


def _mp_body(h_hbm, swd_hbm, w_hbm, zeros_hbm, out_hbm,
             swd0, swd1, w0, w1, r0, r1, r2, r3,
             sem0, sem1, sem2, sem3,
             d0, d1, d2, d3, ssem0, ssem1, ssem2, ssem3, acc_sh):
    c = lax.axis_index("c")
    s = lax.axis_index("s")
    wid = s * _NC + c
    bbase = wid * _BPW

    pltpu.sync_copy(zeros_hbm, acc_sh.at[pl.ds(s * _ZR, _ZR)])
    plsc.subcore_barrier()

    rbufs = (r0, r1, r2, r3)
    sems = (sem0, sem1, sem2, sem3)
    dbufs = (d0, d1, d2, d3)
    ssems = (ssem0, ssem1, ssem2, ssem3)
    sbufs = (swd0, swd1)
    wbufs = (w0, w1)

    def src_ref(sb, k):
        return h_hbm.at[sb.at[pl.ds(k * _CHUNK, _CHUNK)]]

    def scat_copy(k):
        return pltpu.make_async_copy(rbufs[k], acc_sh.at[dbufs[k]], ssems[k])

    def load_block(bi, q):
        pltpu.sync_copy(swd_hbm.at[bbase + bi], sbufs[q])
        pltpu.sync_copy(w_hbm.at[pl.ds((bbase + bi) * _BE, _BE)], wbufs[q])

    load_block(0, 0)
    for k in range(2):
        pltpu.async_copy(src_ref(swd0, k), rbufs[k], sems[k])
    load_block(1, 1)

    def body(i, carry):
        for q in range(2):
            b = 2 * i + q
            sb = sbufs[q]
            nb = sbufs[1 - q]
            for k in range(_RING):
                j = _RING * b + k
                rows_v = rbufs[k]
                pltpu.make_async_copy(src_ref(sb, k), rows_v, sems[k]).wait()
                _scale_chunk(rows_v, wbufs[q], k)
                for g in range(_CHUNK // 16):
                    sl = pl.ds(g * 16, 16)
                    dbufs[k][sl] = sb[pl.ds(_BE + k * _CHUNK + g * 16, 16)]
                pltpu.async_copy(rows_v, acc_sh.at[dbufs[k]], ssems[k],
                                 add=True)

                kc = (k + 2) % 4
                gb, ck = (sb, k + 2) if k < 2 else (nb, k - 2)

                @pl.when((j + 2 < _CPT) & (j >= 2))
                def _():
                    scat_copy(kc).wait()

                @pl.when(j + 2 < _CPT)
                def _():
                    pltpu.async_copy(src_ref(gb, ck), rbufs[kc], sems[kc])

            @pl.when(b + 2 < _BPW)
            def _():
                load_block(b + 2, q)
        return carry

    lax.fori_loop(0, _BPW // 2, body, 0)
    for k in range(_RING):
        scat_copy(k).wait()
    plsc.subcore_barrier()
    pltpu.sync_copy(acc_sh.at[pl.ds(s * _WR, _WR)],
                    out_hbm.at[c].at[pl.ds(s * _WR, _WR)])


@functools.cache
def _mp_builder():
    return functools.partial(
        pl.kernel,
        out_type=jax.ShapeDtypeStruct((_NC, _NPAD, _D), jnp.float32),
        mesh=plsc.VectorSubcoreMesh(core_axis_name="c", subcore_axis_name="s",
                                    num_cores=_NC, num_subcores=_NS),
        scratch_types=[
            pltpu.VMEM((2 * _BE,), jnp.int32),
            pltpu.VMEM((2 * _BE,), jnp.int32),
            pltpu.VMEM((_BE,), jnp.float32),
            pltpu.VMEM((_BE,), jnp.float32),
            pltpu.VMEM((_CHUNK, _D), jnp.float32),
            pltpu.VMEM((_CHUNK, _D), jnp.float32),
            pltpu.VMEM((_CHUNK, _D), jnp.float32),
            pltpu.VMEM((_CHUNK, _D), jnp.float32),
            pltpu.SemaphoreType.DMA,
            pltpu.SemaphoreType.DMA,
            pltpu.SemaphoreType.DMA,
            pltpu.SemaphoreType.DMA,
            pltpu.VMEM((_CHUNK,), jnp.int32),
            pltpu.VMEM((_CHUNK,), jnp.int32),
            pltpu.VMEM((_CHUNK,), jnp.int32),
            pltpu.VMEM((_CHUNK,), jnp.int32),
            pltpu.SemaphoreType.DMA,
            pltpu.SemaphoreType.DMA,
            pltpu.SemaphoreType.DMA,
            pltpu.SemaphoreType.DMA,
            pltpu.VMEM_SHARED((_NPAD, _D), jnp.float32),
        ],
    )(_mp_body)


def _mlp_body(heps_ref, h_ref, hn_ref, w1_ref, b1_ref, g1_ref, be1_ref,
              w2_ref, b2_ref, g2_ref, be2_ref, gid_ref,
              hout_ref, hg_ref, t_scr, u_scr, st1, st2):
    p = pl.program_id(0)
    i = pl.program_id(1)

    @pl.when((p == 0) & (i == 0))
    def _():
        st1[...] = jnp.zeros_like(st1)
        st2[...] = jnp.zeros_like(st2)

    @pl.when(p == 0)
    def _():
        a = h_ref[...] * heps_ref[0, 0] + hn_ref[0] + hn_ref[1]
        t = jnp.dot(a, w1_ref[...], preferred_element_type=jnp.float32)
        t = t + b1_ref[...]
        t_scr[i] = t
        st1[0:1] += jnp.sum(t, axis=0, keepdims=True)
        st1[1:2] += jnp.sum(t * t, axis=0, keepdims=True)

    @pl.when(p == 1)
    def _():
        mean = st1[0:1] * (1.0 / _N)
        var = st1[1:2] * (1.0 / _N) - mean * mean
        scale = g1_ref[...] * lax.rsqrt(var + _BNEPS)
        shift = be1_ref[...] - mean * scale
        r = jnp.maximum(t_scr[i] * scale + shift, 0.0)
        u = jnp.dot(r, w2_ref[...], preferred_element_type=jnp.float32)
        u = u + b2_ref[...]
        u_scr[i] = u
        st2[0:1] += jnp.sum(u, axis=0, keepdims=True)
        st2[1:2] += jnp.sum(u * u, axis=0, keepdims=True)

    @pl.when(p == 2)
    def _():
        mean = st2[0:1] * (1.0 / _N)
        var = st2[1:2] * (1.0 / _N) - mean * mean
        scale = g2_ref[...] * lax.rsqrt(var + _BNEPS)
        shift = be2_ref[...] - mean * scale
        hcur = jnp.maximum(u_scr[i] * scale + shift, 0.0)
        hout_ref[...] = hcur
        oh = (lax.broadcasted_iota(jnp.int32, (_G, _BLK), 0)
              == gid_ref[0]).astype(jnp.float32)

        @pl.when(i == 0)
        def _():
            hg_ref[...] = jnp.zeros_like(hg_ref)

        hg_ref[...] += jnp.dot(oh, hcur, preferred_element_type=jnp.float32)


def _mlp(heps, h, hn, layer, gid3):
    row = pl.BlockSpec((1, _D), lambda p, i: (0, 0))
    return pl.pallas_call(
        _mlp_body,
        grid=(3, _NBLK),
        in_specs=[
            pl.BlockSpec(memory_space=pltpu.SMEM),
            pl.BlockSpec((_BLK, _D), lambda p, i: (i, 0)),
            pl.BlockSpec((_NC, _BLK, _D), lambda p, i: (0, i, 0)),
            pl.BlockSpec((_D, _D), lambda p, i: (0, 0)),
            row, row, row,
            pl.BlockSpec((_D, _D), lambda p, i: (0, 0)),
            row, row, row,
            pl.BlockSpec((1, 1, _BLK), lambda p, i: (i, 0, 0)),
        ],
        out_specs=[
            pl.BlockSpec((_BLK, _D), lambda p, i: (i, 0)),
            pl.BlockSpec((_G, _D), lambda p, i: (0, 0)),
        ],
        out_shape=[
            jax.ShapeDtypeStruct((_N, _D), jnp.float32),
            jax.ShapeDtypeStruct((_G, _D), jnp.float32),
        ],
        scratch_shapes=[
            pltpu.VMEM((_NBLK, _BLK, _D), jnp.float32),
            pltpu.VMEM((_NBLK, _BLK, _D), jnp.float32),
            pltpu.VMEM((8, _D), jnp.float32),
            pltpu.VMEM((8, _D), jnp.float32),
        ],
    )(heps, h, hn, layer["W1"], layer["b1"].reshape(1, _D),
      layer["bn1_gamma"].reshape(1, _D), layer["bn1_beta"].reshape(1, _D),
      layer["W2"], layer["b2"].reshape(1, _D),
      layer["bn_gamma"].reshape(1, _D), layer["bn_beta"].reshape(1, _D),
      gid3)


def _p0_body(gid_ref, x_ref, hg_ref):
    i = pl.program_id(0)
    oh = (lax.broadcasted_iota(jnp.int32, (_G, _BLK), 0)
          == gid_ref[0]).astype(jnp.float32)
    hgc = jnp.dot(oh, x_ref[...], preferred_element_type=jnp.float32)

    @pl.when(i == 0)
    def _():
        hg_ref[...] = jnp.zeros_like(hg_ref)

    hg_ref[...] += hgc


def _p0(gid3, x):
    return pl.pallas_call(
        _p0_body,
        grid=(_NBLK,),
        in_specs=[
            pl.BlockSpec((1, 1, _BLK), lambda i: (i, 0, 0)),
            pl.BlockSpec((_BLK, _D), lambda i: (i, 0)),
        ],
        out_specs=pl.BlockSpec((_G, _D), lambda i: (0, 0)),
        out_shape=jax.ShapeDtypeStruct((_G, _D), jnp.float32),
    )(gid3, x)


def _sk_body(hg_ref, pw_ref, pb_ref, score_ref):
    acc = jnp.zeros((_G, _OUT), jnp.float32)
    for l in range(_LAYERS):
        acc = acc + jnp.dot(hg_ref[l], pw_ref[l],
                            preferred_element_type=jnp.float32) + pb_ref[l]
    score_ref[...] = acc


def _sk(hg_all, pw, pb):
    return pl.pallas_call(
        _sk_body,
        out_shape=jax.ShapeDtypeStruct((_G, _OUT), jnp.float32),
    )(hg_all, pw, pb)


@jax.jit
def kernel(x, edge_index, edge_weight, graph_ids, params):
    pad = _EPAD - _E
    src = jnp.concatenate([edge_index[0], jnp.zeros((pad,), jnp.int32)])
    dst = jnp.concatenate([edge_index[1], jnp.zeros((pad,), jnp.int32)])
    w = jnp.concatenate([edge_weight, jnp.zeros((pad,), jnp.float32)])
    swd = jnp.concatenate([src.reshape(_NW * _BPW, _BE),
                          dst.reshape(_NW * _BPW, _BE)], axis=1)
    zeros_hbm = jnp.zeros((_ZR, _D), jnp.float32)
    gid3 = graph_ids.reshape(_NBLK, 1, _BLK)

    hgs = [_p0(gid3, x)]
    h = x
    for l in range(_LAYERS - 1):
        layer = params["layers"][l]
        hn = _mp_builder()(h, swd, w, zeros_hbm)
        heps = (1.0 + params["eps"][l]).reshape(1, 1)
        h, hg = _mlp(heps, h, hn, layer, gid3)
        hgs.append(hg)

    hg_all = jnp.stack(hgs)
    pw = jnp.stack(params["pred_W"])
    pb = jnp.stack([b.reshape(1, _OUT) for b in params["pred_b"]])
    return _sk(hg_all, pw, pb)


# speedup vs baseline: 1.0391x; 1.0066x over previous
import functools

import jax
import jax.numpy as jnp
from jax import lax
from jax.experimental import pallas as pl
from jax.experimental.pallas import tpu as pltpu
from jax.experimental.pallas import tpu_sc as plsc

_N = 10000
_E = 320000
_G = 64
_D = 128
_OUT = 16
_LAYERS = 5
_BNEPS = 1e-5

_NC, _NS = 2, 16
_NW = _NC * _NS
_CHUNK = 80
_CPT = 128
_EPAD = _NW * _CPT * _CHUNK
_ZR = 640
_NPAD = _NW * _ZR // _NC
_WR = _ZR

_RING = 4
_BPW = _CPT // _RING
_BE = _RING * _CHUNK

_BLK = 1000
_NBLK = _N // _BLK


def _scale_chunk(rows_v, w_v, k):
    for g in range(_CHUNK // 16):
        wv = w_v[pl.ds(k * _CHUNK + g * 16, 16)]
        wes = [wv[e] for e in range(16)]
        for e in range(16):
            r = g * 16 + e
            for f in range(_D // 16):
                sl = pl.ds(f * 16, 16)
                rows_v[r, sl] = rows_v[r, sl] * wes[e]


def _mp_body(h_hbm, swd_hbm, w_hbm, zeros_hbm, out_hbm,
             swd0, swd1, w0, w1, r0, r1, r2, r3,
             sem0, sem1, sem2, sem3,
             d0, d1, d2, d3, ssem0, ssem1, ssem2, ssem3, acc_sh):
    c = lax.axis_index("c")
    s = lax.axis_index("s")
    wid = s * _NC + c
    bbase = wid * _BPW

    pltpu.sync_copy(zeros_hbm, acc_sh.at[pl.ds(s * _ZR, _ZR)])
    plsc.subcore_barrier()

    rbufs = (r0, r1, r2, r3)
    sems = (sem0, sem1, sem2, sem3)
    dbufs = (d0, d1, d2, d3)
    ssems = (ssem0, ssem1, ssem2, ssem3)
    sbufs = (swd0, swd1)
    wbufs = (w0, w1)

    def src_ref(sb, k):
        return h_hbm.at[sb.at[pl.ds(k * _CHUNK, _CHUNK)]]

    def scat_copy(k):
        return pltpu.make_async_copy(rbufs[k], acc_sh.at[dbufs[k]], ssems[k])

    def load_block(bi, q):
        pltpu.sync_copy(swd_hbm.at[bbase + bi], sbufs[q])
        pltpu.sync_copy(w_hbm.at[pl.ds((bbase + bi) * _BE, _BE)], wbufs[q])

    load_block(0, 0)
    for k in range(2):
        pltpu.async_copy(src_ref(swd0, k), rbufs[k], sems[k])
    load_block(1, 1)

    def body(i, carry):
        for q in range(2):
            b = 2 * i + q
            sb = sbufs[q]
            nb = sbufs[1 - q]
            for k in range(_RING):
                j = _RING * b + k
                rows_v = rbufs[k]
                pltpu.make_async_copy(src_ref(sb, k), rows_v, sems[k]).wait()
                _scale_chunk(rows_v, wbufs[q], k)
                for g in range(_CHUNK // 16):
                    sl = pl.ds(g * 16, 16)
                    dbufs[k][sl] = sb[pl.ds(_BE + k * _CHUNK + g * 16, 16)]
                pltpu.async_copy(rows_v, acc_sh.at[dbufs[k]], ssems[k],
                                 add=True)

                kc = (k + 2) % 4
                gb, ck = (sb, k + 2) if k < 2 else (nb, k - 2)

                @pl.when((j + 2 < _CPT) & (j >= 2))
                def _():
                    scat_copy(kc).wait()

                @pl.when(j + 2 < _CPT)
                def _():
                    pltpu.async_copy(src_ref(gb, ck), rbufs[kc], sems[kc])

            @pl.when(b + 2 < _BPW)
            def _():
                load_block(b + 2, q)
        return carry

    lax.fori_loop(0, _BPW // 2, body, 0)
    for k in range(_RING):
        scat_copy(k).wait()
    plsc.subcore_barrier()
    pltpu.sync_copy(acc_sh.at[pl.ds(s * _WR, _WR)],
                    out_hbm.at[c].at[pl.ds(s * _WR, _WR)])


@functools.cache
def _mp_builder():
    return functools.partial(
        pl.kernel,
        out_type=jax.ShapeDtypeStruct((_NC, _NPAD, _D), jnp.float32),
        mesh=plsc.VectorSubcoreMesh(core_axis_name="c", subcore_axis_name="s",
                                    num_cores=_NC, num_subcores=_NS),
        scratch_types=[
            pltpu.VMEM((2 * _BE,), jnp.int32),
            pltpu.VMEM((2 * _BE,), jnp.int32),
            pltpu.VMEM((_BE,), jnp.float32),
            pltpu.VMEM((_BE,), jnp.float32),
            pltpu.VMEM((_CHUNK, _D), jnp.float32),
            pltpu.VMEM((_CHUNK, _D), jnp.float32),
            pltpu.VMEM((_CHUNK, _D), jnp.float32),
            pltpu.VMEM((_CHUNK, _D), jnp.float32),
            pltpu.SemaphoreType.DMA,
            pltpu.SemaphoreType.DMA,
            pltpu.SemaphoreType.DMA,
            pltpu.SemaphoreType.DMA,
            pltpu.VMEM((_CHUNK,), jnp.int32),
            pltpu.VMEM((_CHUNK,), jnp.int32),
            pltpu.VMEM((_CHUNK,), jnp.int32),
            pltpu.VMEM((_CHUNK,), jnp.int32),
            pltpu.SemaphoreType.DMA,
            pltpu.SemaphoreType.DMA,
            pltpu.SemaphoreType.DMA,
            pltpu.SemaphoreType.DMA,
            pltpu.VMEM_SHARED((_NPAD, _D), jnp.float32),
        ],
    )(_mp_body)


def _mlp_body(heps_ref, h_ref, hn_ref, w1_ref, b1_ref, g1_ref, be1_ref,
              w2_ref, b2_ref, g2_ref, be2_ref, gid_ref,
              hout_ref, hg_ref, hgin_ref, t_scr, u_scr, st1, st2):
    p = pl.program_id(0)
    i = pl.program_id(1)

    @pl.when((p == 0) & (i == 0))
    def _():
        st1[...] = jnp.zeros_like(st1)
        st2[...] = jnp.zeros_like(st2)

    @pl.when(p == 0)
    def _():
        hin = h_ref[...]
        a = hin * heps_ref[0, 0] + hn_ref[0] + hn_ref[1]
        t = jnp.dot(a, w1_ref[...], preferred_element_type=jnp.float32)
        t = t + b1_ref[...]
        t_scr[i] = t
        st1[0:1] += jnp.sum(t, axis=0, keepdims=True)
        st1[1:2] += jnp.sum(t * t, axis=0, keepdims=True)
        oh = (lax.broadcasted_iota(jnp.int32, (_G, _BLK), 0)
              == gid_ref[0]).astype(jnp.float32)

        @pl.when(i == 0)
        def _():
            hgin_ref[...] = jnp.zeros_like(hgin_ref)

        hgin_ref[...] += jnp.dot(oh, hin, preferred_element_type=jnp.float32)

    @pl.when(p == 1)
    def _():
        mean = st1[0:1] * (1.0 / _N)
        var = st1[1:2] * (1.0 / _N) - mean * mean
        scale = g1_ref[...] * lax.rsqrt(var + _BNEPS)
        shift = be1_ref[...] - mean * scale
        r = jnp.maximum(t_scr[i] * scale + shift, 0.0)
        u = jnp.dot(r, w2_ref[...], preferred_element_type=jnp.float32)
        u = u + b2_ref[...]
        u_scr[i] = u
        st2[0:1] += jnp.sum(u, axis=0, keepdims=True)
        st2[1:2] += jnp.sum(u * u, axis=0, keepdims=True)

    @pl.when(p == 2)
    def _():
        mean = st2[0:1] * (1.0 / _N)
        var = st2[1:2] * (1.0 / _N) - mean * mean
        scale = g2_ref[...] * lax.rsqrt(var + _BNEPS)
        shift = be2_ref[...] - mean * scale
        hcur = jnp.maximum(u_scr[i] * scale + shift, 0.0)
        hout_ref[...] = hcur
        oh = (lax.broadcasted_iota(jnp.int32, (_G, _BLK), 0)
              == gid_ref[0]).astype(jnp.float32)

        @pl.when(i == 0)
        def _():
            hg_ref[...] = jnp.zeros_like(hg_ref)

        hg_ref[...] += jnp.dot(oh, hcur, preferred_element_type=jnp.float32)


def _mlp(heps, h, hn, layer, gid3):
    row = pl.BlockSpec((1, _D), lambda p, i: (0, 0))
    return pl.pallas_call(
        _mlp_body,
        grid=(3, _NBLK),
        in_specs=[
            pl.BlockSpec(memory_space=pltpu.SMEM),
            pl.BlockSpec((_BLK, _D), lambda p, i: (i, 0)),
            pl.BlockSpec((_NC, _BLK, _D), lambda p, i: (0, i, 0)),
            pl.BlockSpec((_D, _D), lambda p, i: (0, 0)),
            row, row, row,
            pl.BlockSpec((_D, _D), lambda p, i: (0, 0)),
            row, row, row,
            pl.BlockSpec((1, 1, _BLK), lambda p, i: (i, 0, 0)),
        ],
        out_specs=[
            pl.BlockSpec((_BLK, _D), lambda p, i: (i, 0)),
            pl.BlockSpec((_G, _D), lambda p, i: (0, 0)),
            pl.BlockSpec((_G, _D), lambda p, i: (0, 0)),
        ],
        out_shape=[
            jax.ShapeDtypeStruct((_N, _D), jnp.float32),
            jax.ShapeDtypeStruct((_G, _D), jnp.float32),
            jax.ShapeDtypeStruct((_G, _D), jnp.float32),
        ],
        scratch_shapes=[
            pltpu.VMEM((_NBLK, _BLK, _D), jnp.float32),
            pltpu.VMEM((_NBLK, _BLK, _D), jnp.float32),
            pltpu.VMEM((8, _D), jnp.float32),
            pltpu.VMEM((8, _D), jnp.float32),
        ],
    )(heps, h, hn, layer["W1"], layer["b1"].reshape(1, _D),
      layer["bn1_gamma"].reshape(1, _D), layer["bn1_beta"].reshape(1, _D),
      layer["W2"], layer["b2"].reshape(1, _D),
      layer["bn_gamma"].reshape(1, _D), layer["bn_beta"].reshape(1, _D),
      gid3)


def _sk_body(hg_ref, pw_ref, pb_ref, score_ref):
    acc = jnp.zeros((_G, _OUT), jnp.float32)
    for l in range(_LAYERS):
        acc = acc + jnp.dot(hg_ref[l], pw_ref[l],
                            preferred_element_type=jnp.float32) + pb_ref[l]
    score_ref[...] = acc


def _sk(hg_all, pw, pb):
    return pl.pallas_call(
        _sk_body,
        out_shape=jax.ShapeDtypeStruct((_G, _OUT), jnp.float32),
    )(hg_all, pw, pb)


@jax.jit
def kernel(x, edge_index, edge_weight, graph_ids, params):
    pad = _EPAD - _E
    src = jnp.concatenate([edge_index[0], jnp.zeros((pad,), jnp.int32)])
    dst = jnp.concatenate([edge_index[1], jnp.zeros((pad,), jnp.int32)])
    w = jnp.concatenate([edge_weight, jnp.zeros((pad,), jnp.float32)])
    swd = jnp.concatenate([src.reshape(_NW * _BPW, _BE),
                          dst.reshape(_NW * _BPW, _BE)], axis=1)
    zeros_hbm = jnp.zeros((_ZR, _D), jnp.float32)
    gid3 = graph_ids.reshape(_NBLK, 1, _BLK)

    hgs = []
    h = x
    for l in range(_LAYERS - 1):
        layer = params["layers"][l]
        hn = _mp_builder()(h, swd, w, zeros_hbm)
        heps = (1.0 + params["eps"][l]).reshape(1, 1)
        h, hg, hgin = _mlp(heps, h, hn, layer, gid3)
        if l == 0:
            hgs.append(hgin)
        hgs.append(hg)

    hg_all = jnp.stack(hgs)
    pw = jnp.stack(params["pred_W"])
    pb = jnp.stack([b.reshape(1, _OUT) for b in params["pred_b"]])
    return _sk(hg_all, pw, pb)
